# initial kernel scaffold (unmeasured)
import jax
import jax.numpy as jnp
from jax import lax
from jax.experimental import pallas as pl
from jax.experimental.pallas import tpu as pltpu

N_DEV = 4
SQ = 1024
SKV = 1024
HQ = 8
DH = 128
DM = HQ * DH
SCALE = 0.08838834764831843
BLK = 64


def _body(x_ref, wq_ref, k_ref, v_ref, wo_ref, out_ref,
          comm_ref, send_sems, recv_sems, cred_sem):
    my = lax.axis_index("i")
    left = lax.rem(my + N_DEV - 1, N_DEV)
    right = lax.rem(my + 1, N_DEV)

    comm_ref[0, :HQ] = k_ref[...]
    comm_ref[0, HQ:] = v_ref[...]

    barrier_sem = pltpu.get_barrier_semaphore()
    for nbr in (left, right):
        pl.semaphore_signal(barrier_sem, inc=1, device_id=(nbr,),
                            device_id_type=pl.DeviceIdType.MESH)
    pl.semaphore_wait(barrier_sem, 2)

    q = lax.dot(x_ref[...], wq_ref[...], preferred_element_type=jnp.float32)
    qh = [(q[:, h * DH:(h + 1) * DH] * SCALE).astype(jnp.bfloat16)
          for h in range(HQ)]

    qi = lax.broadcasted_iota(jnp.int32, (SQ, SKV), 0)
    kj = lax.broadcasted_iota(jnp.int32, (SQ, SKV), 1)
    qb = (my * SQ + qi) // BLK

    acc = [jnp.zeros((SQ, DH), jnp.float32) for _ in range(HQ)]
    lsum = [jnp.zeros((SQ, 1), jnp.float32) for _ in range(HQ)]

    for h in range(N_DEV):
        slot = h % 2
        if 0 < h < N_DEV - 1:
            pl.semaphore_wait(cred_sem, 1)
        if h < N_DEV - 1:
            rdma = pltpu.make_async_remote_copy(
                src_ref=comm_ref.at[slot],
                dst_ref=comm_ref.at[1 - slot],
                send_sem=send_sems.at[slot],
                recv_sem=recv_sems.at[1 - slot],
                device_id=(right,),
                device_id_type=pl.DeviceIdType.MESH,
            )
            rdma.start()

        src = lax.rem(my - h + N_DEV, N_DEV)
        kb = (src * SKV + kj) // BLK
        mask = (qb == kb) | (kb == 0) | (lax.rem(qb + kb, 3) == 0)
        for hd in range(HQ):
            kmat = comm_ref[slot, hd]
            vmat = comm_ref[slot, HQ + hd]
            s = lax.dot_general(qh[hd], kmat, (((1,), (1,)), ((), ())),
                                preferred_element_type=jnp.float32)
            w = jnp.where(mask, jnp.exp(s), 0.0)
            lsum[hd] = lsum[hd] + jnp.sum(w, axis=1, keepdims=True)
            acc[hd] = acc[hd] + lax.dot(w.astype(jnp.bfloat16), vmat,
                                        preferred_element_type=jnp.float32)

        if h < N_DEV - 1:
            rdma.wait()
            pl.semaphore_signal(cred_sem, inc=1, device_id=(left,),
                                device_id_type=pl.DeviceIdType.MESH)

    ctx = jnp.concatenate(
        [(acc[hd] / lsum[hd]).astype(jnp.bfloat16) for hd in range(HQ)],
        axis=1)
    out_ref[...] = lax.dot(ctx, wo_ref[...], preferred_element_type=jnp.float32)


def kernel(x, Wq, K_ext, V_ext, Wo):
    xb = x[0].astype(jnp.bfloat16)
    wq = Wq.astype(jnp.bfloat16)
    wo = Wo.astype(jnp.bfloat16)
    k2 = jnp.transpose(K_ext[0], (1, 0, 2)).astype(jnp.bfloat16)
    v2 = jnp.transpose(V_ext[0], (1, 0, 2)).astype(jnp.bfloat16)

    out = pl.pallas_call(
        _body,
        out_shape=jax.ShapeDtypeStruct((SQ, DM), jnp.float32),
        in_specs=[pl.BlockSpec(memory_space=pltpu.VMEM)] * 5,
        out_specs=pl.BlockSpec(memory_space=pltpu.VMEM),
        scratch_shapes=[
            pltpu.VMEM((2, 2 * HQ, SKV, DH), jnp.bfloat16),
            pltpu.SemaphoreType.DMA((2,)),
            pltpu.SemaphoreType.DMA((2,)),
            pltpu.SemaphoreType.REGULAR,
        ],
        compiler_params=pltpu.CompilerParams(collective_id=0),
    )(xb, wq, k2, v2, wo)
    return out[None]


# baseline (device time: 228587 ns/iter reference)
import jax
import jax.numpy as jnp
from jax import lax
from jax.experimental import pallas as pl
from jax.experimental.pallas import tpu as pltpu

N_DEV = 4
SQ = 1024
SKV = 1024
HQ = 8
DH = 128
DM = HQ * DH
SCALE = 0.08838834764831843
BLK = 64


def _body(x_ref, wq_ref, k_ref, v_ref, wo_ref, out_ref,
          comm_ref, q_ref, acc_ref, l_ref, send_sems, recv_sems):
    my = lax.axis_index("i")
    left = lax.rem(my + N_DEV - 1, N_DEV)
    right = lax.rem(my + 1, N_DEV)

    comm_ref[0, :HQ] = k_ref[...]
    comm_ref[0, HQ:] = v_ref[...]

    barrier_sem = pltpu.get_barrier_semaphore()
    for nbr in (left, right):
        pl.semaphore_signal(barrier_sem, inc=1, device_id=(nbr,),
                            device_id_type=pl.DeviceIdType.MESH)
    pl.semaphore_wait(barrier_sem, 2)

    def qproj(hd, carry):
        c = hd * DH
        qh = lax.dot(x_ref[...], wq_ref[:, pl.ds(c, DH)],
                     preferred_element_type=jnp.float32)
        q_ref[:, pl.ds(c, DH)] = (qh * SCALE).astype(jnp.bfloat16)
        return carry
    lax.fori_loop(0, HQ, qproj, 0)

    acc_ref[...] = jnp.zeros((SQ, DM), jnp.float32)
    l_ref[...] = jnp.zeros((HQ, SQ, 1), jnp.float32)

    qbv = (my * SQ + lax.broadcasted_iota(jnp.int32, (SQ, 1), 0)) // BLK

    def accumulate(slot, src):
        kbv = (src * SKV + lax.broadcasted_iota(jnp.int32, (1, SKV), 1)) // BLK
        mask = (qbv == kbv) | (kbv == 0) | (lax.rem(qbv + kbv, 3) == 0)

        def head_step(hd, carry):
            c = hd * DH
            s = lax.dot_general(q_ref[:, pl.ds(c, DH)], comm_ref[slot, hd],
                                (((1,), (1,)), ((), ())),
                                preferred_element_type=jnp.float32)
            w = jnp.where(mask, jnp.exp(s), 0.0)
            l_ref[hd] = l_ref[hd] + jnp.sum(w, axis=1, keepdims=True)
            acc_ref[:, pl.ds(c, DH)] = acc_ref[:, pl.ds(c, DH)] + lax.dot(
                w.astype(jnp.bfloat16), comm_ref[slot, HQ + hd],
                preferred_element_type=jnp.float32)
            return carry
        lax.fori_loop(0, HQ, head_step, 0)

    accumulate(0, my)
    for h in range(N_DEV - 1):
        slot = h % 2
        rdma = pltpu.make_async_remote_copy(
            src_ref=comm_ref.at[slot],
            dst_ref=comm_ref.at[1 - slot],
            send_sem=send_sems.at[slot],
            recv_sem=recv_sems.at[1 - slot],
            device_id=(right,),
            device_id_type=pl.DeviceIdType.MESH,
        )
        rdma.start()
        rdma.wait()
        accumulate(1 - slot, lax.rem(my - h - 1 + N_DEV, N_DEV))

    def norm(hd, carry):
        c = hd * DH
        q_ref[:, pl.ds(c, DH)] = (
            acc_ref[:, pl.ds(c, DH)] / l_ref[hd]).astype(jnp.bfloat16)
        return carry
    lax.fori_loop(0, HQ, norm, 0)

    out_ref[...] = lax.dot(q_ref[...], wo_ref[...],
                           preferred_element_type=jnp.float32)


def kernel(x, Wq, K_ext, V_ext, Wo):
    xb = x[0].astype(jnp.bfloat16)
    wq = Wq.astype(jnp.bfloat16)
    wo = Wo.astype(jnp.bfloat16)
    k2 = jnp.transpose(K_ext[0], (1, 0, 2)).astype(jnp.bfloat16)
    v2 = jnp.transpose(V_ext[0], (1, 0, 2)).astype(jnp.bfloat16)

    out = pl.pallas_call(
        _body,
        out_shape=jax.ShapeDtypeStruct((SQ, DM), jnp.float32),
        in_specs=[pl.BlockSpec(memory_space=pltpu.VMEM)] * 5,
        out_specs=pl.BlockSpec(memory_space=pltpu.VMEM),
        scratch_shapes=[
            pltpu.VMEM((2, 2 * HQ, SKV, DH), jnp.bfloat16),
            pltpu.VMEM((SQ, DM), jnp.bfloat16),
            pltpu.VMEM((SQ, DM), jnp.float32),
            pltpu.VMEM((HQ, SQ, 1), jnp.float32),
            pltpu.SemaphoreType.DMA((2,)),
            pltpu.SemaphoreType.DMA((2,)),
        ],
        compiler_params=pltpu.CompilerParams(collective_id=0),
    )(xb, wq, k2, v2, wo)
    return out[None]


# device time: 189457 ns/iter; 1.2065x vs baseline; 1.2065x over previous
import jax
import jax.numpy as jnp
from jax import lax
from jax.experimental import pallas as pl
from jax.experimental.pallas import tpu as pltpu

N_DEV = 4
SQ = 1024
SKV = 1024
HQ = 8
DH = 128
DM = HQ * DH
SCALE = 0.08838834764831843
BLK = 64


def _body(x_ref, wq_ref, k_ref, v_ref, wo_ref, out_ref,
          comm_ref, q_ref, acc_ref, l_ref, send_sems, recv_sems):
    my = lax.axis_index("i")
    left = lax.rem(my + N_DEV - 1, N_DEV)
    right = lax.rem(my + 1, N_DEV)

    comm_ref[0, :HQ] = k_ref[...]
    comm_ref[0, HQ:] = v_ref[...]

    barrier_sem = pltpu.get_barrier_semaphore()
    for nbr in (left, right):
        pl.semaphore_signal(barrier_sem, inc=1, device_id=(nbr,),
                            device_id_type=pl.DeviceIdType.MESH)
    pl.semaphore_wait(barrier_sem, 2)

    def qproj(hd, carry):
        c = hd * DH
        qh = lax.dot(x_ref[...], wq_ref[:, pl.ds(c, DH)],
                     preferred_element_type=jnp.float32)
        q_ref[:, pl.ds(c, DH)] = (qh * SCALE).astype(jnp.bfloat16)
        return carry
    lax.fori_loop(0, HQ, qproj, 0)

    acc_ref[...] = jnp.zeros((SQ, DM), jnp.float32)
    l_ref[...] = jnp.zeros((HQ, SQ, 1), jnp.float32)

    qbv = (my * SQ + lax.broadcasted_iota(jnp.int32, (SQ, 1), 0)) // BLK

    KVH = SKV // 2

    def accumulate(slot, src):
        def head_step(hd, carry):
            c = hd * DH
            qs = q_ref[:, pl.ds(c, DH)]
            for half in range(2):
                o = half * KVH
                kbv = (src * SKV + o
                       + lax.broadcasted_iota(jnp.int32, (1, KVH), 1)) // BLK
                mask = (qbv == kbv) | (kbv == 0) | (lax.rem(qbv + kbv, 3) == 0)
                s = lax.dot_general(qs, comm_ref[slot, hd, pl.ds(o, KVH)],
                                    (((1,), (1,)), ((), ())),
                                    preferred_element_type=jnp.float32)
                w = jnp.where(mask, jnp.exp(s), 0.0)
                l_ref[hd] = l_ref[hd] + jnp.sum(w, axis=1, keepdims=True)
                acc_ref[:, pl.ds(c, DH)] = acc_ref[:, pl.ds(c, DH)] + lax.dot(
                    w.astype(jnp.bfloat16),
                    comm_ref[slot, HQ + hd, pl.ds(o, KVH)],
                    preferred_element_type=jnp.float32)
            return carry
        lax.fori_loop(0, HQ, head_step, 0)

    for h in range(N_DEV - 1):
        rdma = pltpu.make_async_remote_copy(
            src_ref=comm_ref.at[h],
            dst_ref=comm_ref.at[h + 1],
            send_sem=send_sems.at[h],
            recv_sem=recv_sems.at[h],
            device_id=(right,),
            device_id_type=pl.DeviceIdType.MESH,
        )
        rdma.start()
        accumulate(h, lax.rem(my - h + N_DEV, N_DEV))
        rdma.wait()
    accumulate(N_DEV - 1, lax.rem(my + 1, N_DEV))

    def norm(hd, carry):
        c = hd * DH
        q_ref[:, pl.ds(c, DH)] = (
            acc_ref[:, pl.ds(c, DH)] / l_ref[hd]).astype(jnp.bfloat16)
        return carry
    lax.fori_loop(0, HQ, norm, 0)

    out_ref[...] = lax.dot(q_ref[...], wo_ref[...],
                           preferred_element_type=jnp.float32)


def kernel(x, Wq, K_ext, V_ext, Wo):
    xb = x[0].astype(jnp.bfloat16)
    wq = Wq.astype(jnp.bfloat16)
    wo = Wo.astype(jnp.bfloat16)
    k2 = jnp.transpose(K_ext[0], (1, 0, 2)).astype(jnp.bfloat16)
    v2 = jnp.transpose(V_ext[0], (1, 0, 2)).astype(jnp.bfloat16)

    out = pl.pallas_call(
        _body,
        out_shape=jax.ShapeDtypeStruct((SQ, DM), jnp.float32),
        in_specs=[pl.BlockSpec(memory_space=pltpu.VMEM)] * 5,
        out_specs=pl.BlockSpec(memory_space=pltpu.VMEM),
        scratch_shapes=[
            pltpu.VMEM((N_DEV, 2 * HQ, SKV, DH), jnp.bfloat16),
            pltpu.VMEM((SQ, DM), jnp.bfloat16),
            pltpu.VMEM((SQ, DM), jnp.float32),
            pltpu.VMEM((HQ, SQ, 1), jnp.float32),
            pltpu.SemaphoreType.DMA((N_DEV - 1,)),
            pltpu.SemaphoreType.DMA((N_DEV - 1,)),
        ],
        compiler_params=pltpu.CompilerParams(collective_id=0),
    )(xb, wq, k2, v2, wo)
    return out[None]


# device time: 116702 ns/iter; 1.9587x vs baseline; 1.6234x over previous
import jax
import jax.numpy as jnp
from jax import lax
from jax.experimental import pallas as pl
from jax.experimental.pallas import tpu as pltpu

N_DEV = 4
SQ = 1024
SKV = 1024
KVH = SKV // 4
HQ = 8
DH = 128
DM = HQ * DH
SCALE = 0.08838834764831843
BLK = 64

MINE, FROM_L, FROM_R, FROM_D = range(4)


def _body(x_ref, wq_ref, k_ref, v_ref, wo_ref, out_ref,
          comm_ref, q_ref, acc_ref, l_ref, bias_ref, send_sems, recv_sems):
    my = lax.axis_index("i")
    left = lax.rem(my + N_DEV - 1, N_DEV)
    right = lax.rem(my + 1, N_DEV)
    diag = lax.rem(my + 2, N_DEV)

    comm_ref[MINE, :HQ] = k_ref[...]
    comm_ref[MINE, HQ:] = v_ref[...]

    barrier_sem = pltpu.get_barrier_semaphore()
    for nbr in (left, right):
        pl.semaphore_signal(barrier_sem, inc=1, device_id=(nbr,),
                            device_id_type=pl.DeviceIdType.MESH)
    pl.semaphore_wait(barrier_sem, 2)

    send_r = pltpu.make_async_remote_copy(
        src_ref=comm_ref.at[MINE], dst_ref=comm_ref.at[FROM_L],
        send_sem=send_sems.at[0], recv_sem=recv_sems.at[0],
        device_id=(right,), device_id_type=pl.DeviceIdType.MESH)
    send_l = pltpu.make_async_remote_copy(
        src_ref=comm_ref.at[MINE], dst_ref=comm_ref.at[FROM_R],
        send_sem=send_sems.at[1], recv_sem=recv_sems.at[1],
        device_id=(left,), device_id_type=pl.DeviceIdType.MESH)
    send_r.start()
    send_l.start()

    def qproj(hd, carry):
        c = hd * DH
        qh = lax.dot(x_ref[...], wq_ref[:, pl.ds(c, DH)],
                     preferred_element_type=jnp.float32)
        q_ref[:, pl.ds(c, DH)] = (qh * SCALE).astype(jnp.bfloat16)
        return carry
    lax.fori_loop(0, HQ, qproj, 0)

    acc_ref[...] = jnp.zeros((SQ, DM), jnp.float32)
    l_ref[...] = jnp.zeros((HQ, SQ, 1), jnp.float32)

    qbv = (my * SQ + lax.broadcasted_iota(jnp.int32, (SQ, 1), 0)) // BLK

    def accumulate(slot, src):
        for bh in range(4):
            bo = bh * KVH
            kbv = (src * SKV + bo
                   + lax.broadcasted_iota(jnp.int32, (1, KVH), 1)) // BLK
            keep = (qbv == kbv) | (kbv == 0) | (lax.rem(qbv + kbv, 3) == 0)
            bias_ref[:, bo:bo + KVH] = jnp.where(keep, 0.0, -40.0)

        def head_step(hd, carry):
            c = hd * DH
            qs = q_ref[:, pl.ds(c, DH)]
            for half in range(4):
                o = half * KVH
                s = lax.dot_general(qs, comm_ref[slot, hd, pl.ds(o, KVH)],
                                    (((1,), (1,)), ((), ())),
                                    preferred_element_type=jnp.float32)
                w = jnp.exp(s + bias_ref[:, o:o + KVH])
                l_ref[hd] = l_ref[hd] + jnp.sum(w, axis=1, keepdims=True)
                acc_ref[:, pl.ds(c, DH)] = acc_ref[:, pl.ds(c, DH)] + lax.dot(
                    w.astype(jnp.bfloat16),
                    comm_ref[slot, HQ + hd, pl.ds(o, KVH)],
                    preferred_element_type=jnp.float32)
            return carry
        lax.fori_loop(0, HQ, head_step, 0)

    accumulate(MINE, my)

    send_r.wait()
    fwd_k = pltpu.make_async_remote_copy(
        src_ref=comm_ref.at[FROM_L, pl.ds(0, HQ)],
        dst_ref=comm_ref.at[FROM_D, pl.ds(0, HQ)],
        send_sem=send_sems.at[2], recv_sem=recv_sems.at[2],
        device_id=(right,), device_id_type=pl.DeviceIdType.MESH)
    fwd_k.start()
    accumulate(FROM_L, left)

    send_l.wait()
    fwd_v = pltpu.make_async_remote_copy(
        src_ref=comm_ref.at[FROM_R, pl.ds(HQ, HQ)],
        dst_ref=comm_ref.at[FROM_D, pl.ds(HQ, HQ)],
        send_sem=send_sems.at[3], recv_sem=recv_sems.at[3],
        device_id=(left,), device_id_type=pl.DeviceIdType.MESH)
    fwd_v.start()
    accumulate(FROM_R, right)

    fwd_k.wait()
    fwd_v.wait()
    accumulate(FROM_D, diag)

    def norm(hd, carry):
        c = hd * DH
        q_ref[:, pl.ds(c, DH)] = (
            acc_ref[:, pl.ds(c, DH)] / l_ref[hd]).astype(jnp.bfloat16)
        return carry
    lax.fori_loop(0, HQ, norm, 0)

    out_ref[...] = lax.dot(q_ref[...], wo_ref[...],
                           preferred_element_type=jnp.float32)


def kernel(x, Wq, K_ext, V_ext, Wo):
    xb = x[0].astype(jnp.bfloat16)
    wq = Wq.astype(jnp.bfloat16)
    wo = Wo.astype(jnp.bfloat16)
    k2 = jnp.transpose(K_ext[0], (1, 0, 2)).astype(jnp.bfloat16)
    v2 = jnp.transpose(V_ext[0], (1, 0, 2)).astype(jnp.bfloat16)

    out = pl.pallas_call(
        _body,
        out_shape=jax.ShapeDtypeStruct((SQ, DM), jnp.float32),
        in_specs=[pl.BlockSpec(memory_space=pltpu.VMEM)] * 5,
        out_specs=pl.BlockSpec(memory_space=pltpu.VMEM),
        scratch_shapes=[
            pltpu.VMEM((N_DEV, 2 * HQ, SKV, DH), jnp.bfloat16),
            pltpu.VMEM((SQ, DM), jnp.bfloat16),
            pltpu.VMEM((SQ, DM), jnp.float32),
            pltpu.VMEM((HQ, SQ, 1), jnp.float32),
            pltpu.VMEM((SQ, SKV), jnp.float32),
            pltpu.SemaphoreType.DMA((4,)),
            pltpu.SemaphoreType.DMA((4,)),
        ],
        compiler_params=pltpu.CompilerParams(collective_id=0),
    )(xb, wq, k2, v2, wo)
    return out[None]
